# baseline (speedup 1.0000x reference)
import functools

import jax
import jax.numpy as jnp
from jax import lax
from jax.experimental import pallas as pl
from jax.experimental.pallas import tpu as pltpu
from jax.experimental.pallas import tpu_sc as plsc

_NUM_E = 8192
_DIM = 32
_ROWS = 16384
_ROW_TILE = 256
_N_TILES = _ROWS // _ROW_TILE
_CHUNK = 4096
_N_CHUNKS = _NUM_E // _CHUNK
_COMMIT = 0.25


def _bf16_round(x):
    return x.astype(jnp.bfloat16).astype(jnp.float32)


def _dist_argmin_body(zt_ref, cbT_ref, c2_ref, idx_ref, loss_ref):
    i = pl.program_id(0)
    zt = zt_ref[0]
    z2 = jnp.sum(zt * zt, axis=0).reshape(_ROW_TILE, 1)
    zb = zt.astype(jnp.bfloat16)
    mm = lax.dot_general(
        zb, cbT_ref[...], (((0,), (0,)), ((), ())),
        preferred_element_type=jnp.float32)
    dist = (z2 - 2.0 * mm) + c2_ref[...]

    d0 = dist[:, :_CHUNK]
    d1 = dist[:, _CHUNK:]
    m0 = jnp.min(d0, axis=1, keepdims=True)
    m1 = jnp.min(d1, axis=1, keepdims=True)
    upd = m1 < _bf16_round(m0)
    acc_exact = jnp.where(upd, m1, m0)
    dsel = jnp.where(upd, d1, d0)
    ids = lax.broadcasted_iota(jnp.int32, dsel.shape, 1)
    li = jnp.min(jnp.where(dsel == acc_exact, ids, _NUM_E), axis=1)
    idx_ref[0, 0, :] = li + jnp.where(upd[:, 0], _CHUNK, 0)

    @pl.when(i == 0)
    def _init():
        loss_ref[...] = jnp.zeros((1, 1), jnp.float32)

    loss_ref[...] += jnp.sum(acc_exact, keepdims=True)


_dist_argmin = pl.pallas_call(
    _dist_argmin_body,
    grid=(_N_TILES,),
    in_specs=[
        pl.BlockSpec((1, _DIM, _ROW_TILE),
                     lambda i: (i // 4, 0, i % 4)),
        pl.BlockSpec((_DIM, _NUM_E), lambda i: (0, 0)),
        pl.BlockSpec((1, _NUM_E), lambda i: (0, 0)),
    ],
    out_specs=[
        pl.BlockSpec((1, 1, _ROW_TILE), lambda i: (i, 0, 0)),
        pl.BlockSpec((1, 1), lambda i: (0, 0)),
    ],
    out_shape=[
        jax.ShapeDtypeStruct((_N_TILES, 1, _ROW_TILE), jnp.int32),
        jax.ShapeDtypeStruct((1, 1), jnp.float32),
    ],
)


_NUM_CORES = 2
_NUM_SUBCORES = 16
_NW = _NUM_CORES * _NUM_SUBCORES
_B_PER_W = _ROWS // _NW
_GATHER_W = 128


@functools.cache
def _make_sc_gather():
    @functools.partial(
        pl.kernel,
        out_type=jax.ShapeDtypeStruct((_ROWS, _GATHER_W), jnp.float32),
        mesh=plsc.VectorSubcoreMesh(core_axis_name="c", subcore_axis_name="s"),
        scratch_types=[
            pltpu.VMEM((_B_PER_W,), jnp.int32),
            pltpu.VMEM((_B_PER_W, _GATHER_W), jnp.float32),
            pltpu.SemaphoreType.DMA,
        ],
    )
    def _sc_gather(table_hbm, idx_hbm, out_hbm, idx_v, rows_v, sem):
        wid = lax.axis_index("s") * _NUM_CORES + lax.axis_index("c")
        base = wid * _B_PER_W
        pltpu.sync_copy(idx_hbm.at[pl.ds(base, _B_PER_W)], idx_v)
        pltpu.async_copy(table_hbm.at[idx_v], rows_v, sem).wait()
        pltpu.sync_copy(rows_v, out_hbm.at[pl.ds(base, _B_PER_W)])

    return _sc_gather


def kernel(z, codebook):
    B, C, H, W = z.shape
    z_cm = z.reshape(B, C, H * W)
    c2 = jnp.sum(codebook ** 2, axis=1)
    cbT_b = codebook.astype(jnp.bfloat16).T
    idx3, loss_sum = _dist_argmin(z_cm, cbT_b, c2.reshape(1, _NUM_E))
    idx = idx3.reshape(_ROWS)
    cb_pad = jnp.pad(codebook, ((0, 0), (0, _GATHER_W - _DIM)))
    q_flat = _make_sc_gather()(cb_pad, idx)[:, :_DIM]
    quantized = jnp.transpose(q_flat.reshape(B, H, W, C), (0, 3, 1, 2))
    loss = loss_sum[0, 0] * ((1.0 + _COMMIT) / float(_ROWS * _DIM))
    quantized_st = z + lax.stop_gradient(quantized - z)
    return (quantized_st, loss)


# speedup vs baseline: 1.0076x; 1.0076x over previous
import functools

import jax
import jax.numpy as jnp
from jax import lax
from jax.experimental import pallas as pl
from jax.experimental.pallas import tpu as pltpu
from jax.experimental.pallas import tpu_sc as plsc

_NUM_E = 8192
_DIM = 32
_ROWS = 16384
_ROW_TILE = 256
_N_TILES = _ROWS // _ROW_TILE
_CHUNK = 4096
_N_CHUNKS = _NUM_E // _CHUNK
_COMMIT = 0.25


def _bf16_round(x):
    return x.astype(jnp.bfloat16).astype(jnp.float32)


def _dist_argmin_body(zt_ref, cbT_ref, c2_ref, idx_ref, loss_ref):
    i = pl.program_id(0)
    zt = zt_ref[...]
    z2 = jnp.sum(zt * zt, axis=1, keepdims=True)
    zb = zt.astype(jnp.bfloat16)
    mm = jnp.dot(zb, cbT_ref[...], preferred_element_type=jnp.float32)
    dist = (z2 - 2.0 * mm) + c2_ref[...]

    d0 = dist[:, :_CHUNK]
    d1 = dist[:, _CHUNK:]
    m0 = jnp.min(d0, axis=1, keepdims=True)
    m1 = jnp.min(d1, axis=1, keepdims=True)
    upd = m1 < _bf16_round(m0)
    acc_exact = jnp.where(upd, m1, m0)
    dsel = jnp.where(upd, d1, d0)
    ids = lax.broadcasted_iota(jnp.int32, dsel.shape, 1)
    li = jnp.min(jnp.where(dsel == acc_exact, ids, _NUM_E), axis=1)
    idx_ref[0, 0, :] = li + jnp.where(upd[:, 0], _CHUNK, 0)

    @pl.when(i == 0)
    def _init():
        loss_ref[...] = jnp.zeros((1, 1), jnp.float32)

    loss_ref[...] += jnp.sum(acc_exact, keepdims=True)


_dist_argmin = pl.pallas_call(
    _dist_argmin_body,
    grid=(_N_TILES,),
    in_specs=[
        pl.BlockSpec((_ROW_TILE, _DIM), lambda i: (i, 0)),
        pl.BlockSpec((_DIM, _NUM_E), lambda i: (0, 0)),
        pl.BlockSpec((1, _NUM_E), lambda i: (0, 0)),
    ],
    out_specs=[
        pl.BlockSpec((1, 1, _ROW_TILE), lambda i: (i, 0, 0)),
        pl.BlockSpec((1, 1), lambda i: (0, 0)),
    ],
    out_shape=[
        jax.ShapeDtypeStruct((_N_TILES, 1, _ROW_TILE), jnp.int32),
        jax.ShapeDtypeStruct((1, 1), jnp.float32),
    ],
)


_NUM_CORES = 2
_NUM_SUBCORES = 16
_NW = _NUM_CORES * _NUM_SUBCORES
_B_PER_W = _ROWS // _NW
_GATHER_W = 128


@functools.cache
def _make_sc_gather():
    @functools.partial(
        pl.kernel,
        out_type=jax.ShapeDtypeStruct((_ROWS, _GATHER_W), jnp.float32),
        mesh=plsc.VectorSubcoreMesh(core_axis_name="c", subcore_axis_name="s"),
        scratch_types=[
            pltpu.VMEM((_B_PER_W,), jnp.int32),
            pltpu.VMEM((_B_PER_W, _GATHER_W), jnp.float32),
            pltpu.SemaphoreType.DMA,
        ],
    )
    def _sc_gather(table_hbm, idx_hbm, out_hbm, idx_v, rows_v, sem):
        wid = lax.axis_index("s") * _NUM_CORES + lax.axis_index("c")
        base = wid * _B_PER_W
        pltpu.sync_copy(idx_hbm.at[pl.ds(base, _B_PER_W)], idx_v)
        pltpu.async_copy(table_hbm.at[idx_v], rows_v, sem).wait()
        pltpu.sync_copy(rows_v, out_hbm.at[pl.ds(base, _B_PER_W)])

    return _sc_gather


def kernel(z, codebook):
    B, C, H, W = z.shape
    z_flattened = jnp.transpose(z, (0, 2, 3, 1))
    flat_z = z_flattened.reshape(-1, C)
    c2 = jnp.sum(codebook ** 2, axis=1)
    cbT_b = codebook.astype(jnp.bfloat16).T
    idx3, loss_sum = _dist_argmin(flat_z, cbT_b, c2.reshape(1, _NUM_E))
    idx = idx3.reshape(_ROWS)
    cb_pad = jnp.pad(codebook, ((0, 0), (0, _GATHER_W - _DIM)))
    q_flat = _make_sc_gather()(cb_pad, idx)[:, :_DIM]
    quantized = jnp.transpose(q_flat.reshape(B, H, W, C), (0, 3, 1, 2))
    loss = loss_sum[0, 0] * ((1.0 + _COMMIT) / float(_ROWS * _DIM))
    quantized_st = z + lax.stop_gradient(quantized - z)
    return (quantized_st, loss)
